# baseline (speedup 1.0000x reference)
import functools

import jax
import jax.numpy as jnp
from jax import lax
from jax.experimental import pallas as pl
from jax.experimental.pallas import tpu as pltpu
from jax.experimental.pallas import tpu_sc as plsc

N = 10000
E = 320000
F_IN = 128
F_OUT = 64

NC = 2
NS = 16
NW = NC * NS
EPW = E // NW
CH = 100
NCHUNK = EPW // CH
SPAN = 632
N_PAD = SPAN * NS
DEG_PAD = 10240
DEG_SPAN = DEG_PAD // NS
ZROWS = 80

_MESH = plsc.VectorSubcoreMesh(core_axis_name="c", subcore_axis_name="s")
_SC_PARAMS = pltpu.CompilerParams(
    use_tc_tiling_on_sc=False, needs_layout_passes=False
)


DEG_W = 16


@functools.partial(
    pl.kernel,
    out_type=jax.ShapeDtypeStruct((NC, DEG_PAD), jnp.float32),
    mesh=_MESH,
    compiler_params=_SC_PARAMS,
    scratch_types=[
        pltpu.VMEM((NCHUNK, CH), jnp.int32),
        pltpu.VMEM((CH, DEG_W), jnp.float32),
        pltpu.VMEM((DEG_SPAN, DEG_W), jnp.float32),
        pltpu.VMEM((DEG_SPAN,), jnp.float32),
        pltpu.VMEM_SHARED((DEG_PAD, DEG_W), jnp.float32),
        pltpu.SemaphoreType.DMA,
    ],
)
def _deg_kernel(e4, degp, dst2, ones_v, degv, deg1, deg_sh, dsem):
    cid = lax.axis_index("c")
    sid = lax.axis_index("s")
    wid = cid * NS + sid

    @pl.loop(0, CH)
    def _fill_ones(r):
        ones_v[r, :] = jnp.ones((DEG_W,), jnp.float32)

    @pl.loop(0, DEG_SPAN)
    def _fill_z(r):
        degv[r, :] = jnp.zeros((DEG_W,), jnp.float32)

    pltpu.sync_copy(degv, deg_sh.at[pl.ds(sid * DEG_SPAN, DEG_SPAN)])
    pltpu.sync_copy(e4.at[1, wid], dst2)
    plsc.subcore_barrier()

    @pl.loop(0, NCHUNK)
    def _accum(j):
        pltpu.async_copy(ones_v, deg_sh.at[dst2.at[j]], dsem, add=True)

    @pl.loop(0, NCHUNK)
    def _drain(j):
        pltpu.make_async_copy(ones_v, deg_sh.at[dst2.at[j]], dsem).wait()

    plsc.subcore_barrier()
    pltpu.sync_copy(deg_sh.at[pl.ds(sid * DEG_SPAN, DEG_SPAN)], degv)

    @pl.loop(0, DEG_SPAN // 16)
    def _compact(g):
        rows = g * 16 + lax.iota(jnp.int32, 16)
        cols = jnp.zeros((16,), jnp.int32)
        deg1[pl.ds(g * 16, 16)] = plsc.load_gather(degv, [rows, cols])

    pltpu.sync_copy(deg1, degp.at[cid, pl.ds(sid * DEG_SPAN, DEG_SPAN)])


NBUF = 10
DEPTH = 5


@functools.partial(
    pl.kernel,
    out_type=jax.ShapeDtypeStruct((NC, N_PAD, F_OUT), jnp.float32),
    mesh=_MESH,
    compiler_params=_SC_PARAMS,
    scratch_types=[
        pltpu.VMEM((NCHUNK, CH), jnp.int32),
        pltpu.VMEM((NCHUNK, CH), jnp.int32),
    ]
    + [pltpu.VMEM((CH, F_OUT), jnp.float32)] * NBUF
    + [
        pltpu.VMEM((ZROWS, F_OUT), jnp.float32),
        pltpu.VMEM_SHARED((N_PAD, F_OUT), jnp.float32),
    ]
    + [pltpu.SemaphoreType.DMA] * (2 * NBUF),
)
def _agg_kernel(hp, e4, out, src2, dst2, *rest):
    bufs = rest[:NBUF]
    zbuf = rest[NBUF]
    agg_sh = rest[NBUF + 1]
    gsem = rest[NBUF + 2:NBUF + 2 + NBUF]
    ssem = rest[NBUF + 2 + NBUF:]
    cid = lax.axis_index("c")
    sid = lax.axis_index("s")
    wid = cid * NS + sid

    @pl.loop(0, ZROWS)
    def _zero(r):
        for c in range(F_OUT // 16):
            zbuf[r, pl.ds(c * 16, 16)] = jnp.zeros((16,), jnp.float32)

    r0 = sid * SPAN
    for t in range(SPAN // ZROWS):
        pltpu.sync_copy(zbuf, agg_sh.at[pl.ds(r0 + t * ZROWS, ZROWS)])
    rem = SPAN - (SPAN // ZROWS) * ZROWS
    pltpu.sync_copy(
        zbuf.at[pl.ds(0, rem)],
        agg_sh.at[pl.ds(r0 + SPAN - rem, rem)],
    )
    pltpu.sync_copy(e4.at[0, wid], src2)
    pltpu.sync_copy(e4.at[1, wid], dst2)
    plsc.subcore_barrier()

    def wait_g(k, b):
        pltpu.make_async_copy(hp.at[src2.at[k]], bufs[b], gsem[b]).wait()

    def fire_s(k, b):
        pltpu.async_copy(bufs[b], agg_sh.at[dst2.at[k]], ssem[b], add=True)

    def wait_s(k, b):
        pltpu.make_async_copy(bufs[b], agg_sh.at[dst2.at[k]], ssem[b]).wait()

    for k in range(DEPTH):
        pltpu.async_copy(hp.at[src2.at[k]], bufs[k % NBUF], gsem[k % NBUF])
    for k in range(DEPTH):
        bb = k % NBUF
        wait_g(k, bb)
        fire_s(k, bb)
        b4 = (k + DEPTH) % NBUF
        pltpu.async_copy(hp.at[src2.at[k + DEPTH]], bufs[b4], gsem[b4])

    @pl.loop(DEPTH, NCHUNK - DEPTH, step=NBUF)
    def _edges(j):
        for u in range(NBUF):
            k = j + u
            bb = (DEPTH + u) % NBUF
            b4 = u % NBUF
            wait_g(k, bb)
            fire_s(k, bb)
            wait_s(k - DEPTH, b4)
            pltpu.async_copy(hp.at[src2.at[k + DEPTH]], bufs[b4], gsem[b4])

    for k in range(NCHUNK - DEPTH, NCHUNK):
        bb = k % NBUF
        wait_g(k, bb)
        fire_s(k, bb)
        wait_s(k - DEPTH, (k + DEPTH) % NBUF)
    for k in range(NCHUNK - DEPTH, NCHUNK):
        wait_s(k, k % NBUF)
    plsc.subcore_barrier()
    pltpu.sync_copy(agg_sh.at[pl.ds(r0, SPAN)], out.at[cid, pl.ds(r0, SPAN)])


_RB = 2000


def _mm_body(x_ref, wi_ref, wr_ref, h_ref, root_ref):
    x = x_ref[...]
    h_ref[...] = jnp.dot(x, wi_ref[...], preferred_element_type=jnp.float32)
    root_ref[...] = jnp.dot(x, wr_ref[...], preferred_element_type=jnp.float32)


def _mm(x, wi, wr):
    grid = (N // _RB,)
    return pl.pallas_call(
        _mm_body,
        grid=grid,
        in_specs=[
            pl.BlockSpec((_RB, F_IN), lambda i: (i, 0)),
            pl.BlockSpec((F_IN, F_OUT), lambda i: (0, 0)),
            pl.BlockSpec((F_IN, F_OUT), lambda i: (0, 0)),
        ],
        out_specs=[
            pl.BlockSpec((_RB, F_OUT), lambda i: (i, 0)),
            pl.BlockSpec((_RB, F_OUT), lambda i: (i, 0)),
        ],
        out_shape=[
            jax.ShapeDtypeStruct((N, F_OUT), jnp.float32),
            jax.ShapeDtypeStruct((N, F_OUT), jnp.float32),
        ],
    )(x, wi, wr)


def kernel(x, edge_index, W_init, W_root, bias):
    e4 = edge_index.reshape(2, NW, NCHUNK, CH)
    degp = _deg_kernel(e4)
    h, root = _mm(x, W_init, W_root)
    deg = degp[0, :N] + degp[1, :N]
    dinv = jnp.where(deg > 0, lax.rsqrt(deg), 0.0)[:, None]
    hp = h * dinv
    p = _agg_kernel(hp, e4)
    return jnp.maximum(dinv * (p[0, :N] + p[1, :N]) + root + bias, 0.0)


# speedup vs baseline: 1.0168x; 1.0168x over previous
import functools

import jax
import jax.numpy as jnp
from jax import lax
from jax.experimental import pallas as pl
from jax.experimental.pallas import tpu as pltpu
from jax.experimental.pallas import tpu_sc as plsc

N = 10000
E = 320000
F_IN = 128
F_OUT = 64

NC = 2
NS = 16
NW = NC * NS
EPW = E // NW
CH = 125
NCHUNK = EPW // CH
SPAN = 632
N_PAD = SPAN * NS
DEG_PAD = 10240
DEG_SPAN = DEG_PAD // NS
ZROWS = 80

_MESH = plsc.VectorSubcoreMesh(core_axis_name="c", subcore_axis_name="s")
_SC_PARAMS = pltpu.CompilerParams(
    use_tc_tiling_on_sc=False, needs_layout_passes=False
)


DEG_W = 16


@functools.partial(
    pl.kernel,
    out_type=jax.ShapeDtypeStruct((NC, DEG_PAD), jnp.float32),
    mesh=_MESH,
    compiler_params=_SC_PARAMS,
    scratch_types=[
        pltpu.VMEM((NCHUNK, CH), jnp.int32),
        pltpu.VMEM((CH, DEG_W), jnp.float32),
        pltpu.VMEM((DEG_SPAN, DEG_W), jnp.float32),
        pltpu.VMEM((DEG_SPAN,), jnp.float32),
        pltpu.VMEM_SHARED((DEG_PAD, DEG_W), jnp.float32),
        pltpu.SemaphoreType.DMA,
    ],
)
def _deg_kernel(e4, degp, dst2, ones_v, degv, deg1, deg_sh, dsem):
    cid = lax.axis_index("c")
    sid = lax.axis_index("s")
    wid = cid * NS + sid

    @pl.loop(0, CH)
    def _fill_ones(r):
        ones_v[r, :] = jnp.ones((DEG_W,), jnp.float32)

    @pl.loop(0, DEG_SPAN)
    def _fill_z(r):
        degv[r, :] = jnp.zeros((DEG_W,), jnp.float32)

    pltpu.sync_copy(degv, deg_sh.at[pl.ds(sid * DEG_SPAN, DEG_SPAN)])
    pltpu.sync_copy(e4.at[1, wid], dst2)
    plsc.subcore_barrier()

    @pl.loop(0, NCHUNK)
    def _accum(j):
        pltpu.async_copy(ones_v, deg_sh.at[dst2.at[j]], dsem, add=True)

    @pl.loop(0, NCHUNK)
    def _drain(j):
        pltpu.make_async_copy(ones_v, deg_sh.at[dst2.at[j]], dsem).wait()

    plsc.subcore_barrier()
    pltpu.sync_copy(deg_sh.at[pl.ds(sid * DEG_SPAN, DEG_SPAN)], degv)

    @pl.loop(0, DEG_SPAN // 16)
    def _compact(g):
        rows = g * 16 + lax.iota(jnp.int32, 16)
        cols = jnp.zeros((16,), jnp.int32)
        deg1[pl.ds(g * 16, 16)] = plsc.load_gather(degv, [rows, cols])

    pltpu.sync_copy(deg1, degp.at[cid, pl.ds(sid * DEG_SPAN, DEG_SPAN)])


NBUF = 8
DEPTH = 4


@functools.partial(
    pl.kernel,
    out_type=jax.ShapeDtypeStruct((NC, N_PAD, F_OUT), jnp.float32),
    mesh=_MESH,
    compiler_params=_SC_PARAMS,
    scratch_types=[
        pltpu.VMEM((NCHUNK, CH), jnp.int32),
        pltpu.VMEM((NCHUNK, CH), jnp.int32),
    ]
    + [pltpu.VMEM((CH, F_OUT), jnp.float32)] * NBUF
    + [
        pltpu.VMEM((ZROWS, F_OUT), jnp.float32),
        pltpu.VMEM_SHARED((N_PAD, F_OUT), jnp.float32),
    ]
    + [pltpu.SemaphoreType.DMA] * (2 * NBUF),
)
def _agg_kernel(hp, e4, out, src2, dst2, *rest):
    bufs = rest[:NBUF]
    zbuf = rest[NBUF]
    agg_sh = rest[NBUF + 1]
    gsem = rest[NBUF + 2:NBUF + 2 + NBUF]
    ssem = rest[NBUF + 2 + NBUF:]
    cid = lax.axis_index("c")
    sid = lax.axis_index("s")
    wid = cid * NS + sid

    @pl.loop(0, ZROWS)
    def _zero(r):
        for c in range(F_OUT // 16):
            zbuf[r, pl.ds(c * 16, 16)] = jnp.zeros((16,), jnp.float32)

    r0 = sid * SPAN
    for t in range(SPAN // ZROWS):
        pltpu.sync_copy(zbuf, agg_sh.at[pl.ds(r0 + t * ZROWS, ZROWS)])
    rem = SPAN - (SPAN // ZROWS) * ZROWS
    pltpu.sync_copy(
        zbuf.at[pl.ds(0, rem)],
        agg_sh.at[pl.ds(r0 + SPAN - rem, rem)],
    )
    pltpu.sync_copy(e4.at[0, wid], src2)
    pltpu.sync_copy(e4.at[1, wid], dst2)
    plsc.subcore_barrier()

    def wait_g(k, b):
        pltpu.make_async_copy(hp.at[src2.at[k]], bufs[b], gsem[b]).wait()

    def fire_s(k, b):
        pltpu.async_copy(bufs[b], agg_sh.at[dst2.at[k]], ssem[b], add=True)

    def wait_s(k, b):
        pltpu.make_async_copy(bufs[b], agg_sh.at[dst2.at[k]], ssem[b]).wait()

    for k in range(DEPTH):
        pltpu.async_copy(hp.at[src2.at[k]], bufs[k % NBUF], gsem[k % NBUF])
    for k in range(DEPTH):
        bb = k % NBUF
        wait_g(k, bb)
        fire_s(k, bb)
        b4 = (k + DEPTH) % NBUF
        pltpu.async_copy(hp.at[src2.at[k + DEPTH]], bufs[b4], gsem[b4])

    @pl.loop(DEPTH, NCHUNK - DEPTH, step=NBUF)
    def _edges(j):
        for u in range(NBUF):
            k = j + u
            bb = (DEPTH + u) % NBUF
            b4 = u % NBUF
            wait_g(k, bb)
            fire_s(k, bb)
            wait_s(k - DEPTH, b4)
            pltpu.async_copy(hp.at[src2.at[k + DEPTH]], bufs[b4], gsem[b4])

    for k in range(NCHUNK - DEPTH, NCHUNK):
        bb = k % NBUF
        wait_g(k, bb)
        fire_s(k, bb)
        wait_s(k - DEPTH, (k + DEPTH) % NBUF)
    for k in range(NCHUNK - DEPTH, NCHUNK):
        wait_s(k, k % NBUF)
    plsc.subcore_barrier()
    pltpu.sync_copy(agg_sh.at[pl.ds(r0, SPAN)], out.at[cid, pl.ds(r0, SPAN)])


_RB = 2000


def _mm_body(x_ref, wi_ref, wr_ref, h_ref, root_ref):
    x = x_ref[...]
    h_ref[...] = jnp.dot(x, wi_ref[...], preferred_element_type=jnp.float32)
    root_ref[...] = jnp.dot(x, wr_ref[...], preferred_element_type=jnp.float32)


def _mm(x, wi, wr):
    grid = (N // _RB,)
    return pl.pallas_call(
        _mm_body,
        grid=grid,
        in_specs=[
            pl.BlockSpec((_RB, F_IN), lambda i: (i, 0)),
            pl.BlockSpec((F_IN, F_OUT), lambda i: (0, 0)),
            pl.BlockSpec((F_IN, F_OUT), lambda i: (0, 0)),
        ],
        out_specs=[
            pl.BlockSpec((_RB, F_OUT), lambda i: (i, 0)),
            pl.BlockSpec((_RB, F_OUT), lambda i: (i, 0)),
        ],
        out_shape=[
            jax.ShapeDtypeStruct((N, F_OUT), jnp.float32),
            jax.ShapeDtypeStruct((N, F_OUT), jnp.float32),
        ],
    )(x, wi, wr)


def kernel(x, edge_index, W_init, W_root, bias):
    e4 = edge_index.reshape(2, NW, NCHUNK, CH)
    degp = _deg_kernel(e4)
    h, root = _mm(x, W_init, W_root)
    deg = degp[0, :N] + degp[1, :N]
    dinv = jnp.where(deg > 0, lax.rsqrt(deg), 0.0)[:, None]
    hp = h * dinv
    p = _agg_kernel(hp, e4)
    return jnp.maximum(dinv * (p[0, :N] + p[1, :N]) + root + bias, 0.0)
